# baseline (speedup 1.0000x reference)
import functools

import jax
import jax.numpy as jnp
from jax import lax
from jax.experimental import pallas as pl
from jax.experimental.pallas import tpu as pltpu
from jax.experimental.pallas import tpu_sc as plsc

_BUF_ROWS = 40
_NBUF = 3


def kernel(x, pe):
    seq_len = x.shape[1]
    n_emb = pe.shape[1]
    info = plsc.get_sparse_core_info()
    nc, ns = info.num_cores, info.num_subcores
    nw = nc * ns
    rows_per_w = seq_len // nw

    sizes = []
    off = 0
    while off < rows_per_w:
        c = min(_BUF_ROWS, rows_per_w - off)
        sizes.append((off, c))
        off += c
    nchunk = len(sizes)

    @functools.partial(
        pl.kernel,
        mesh=plsc.VectorSubcoreMesh(core_axis_name="c", subcore_axis_name="s"),
        out_type=jax.ShapeDtypeStruct((seq_len, n_emb), pe.dtype),
        scratch_types=[
            pltpu.VMEM_SHARED((ns, _NBUF, _BUF_ROWS, n_emb), pe.dtype),
            pltpu.SemaphoreType.DMA((_NBUF,)),
            pltpu.SemaphoreType.DMA((_NBUF,)),
        ],
    )
    def copy_k(pe_hbm, out_hbm, buf, gsem, ssem):
        sid = lax.axis_index("s")
        wid = sid * nc + lax.axis_index("c")
        base = wid * rows_per_w

        def src(i):
            o, c = sizes[i]
            return pe_hbm.at[pl.ds(base + o, c)]

        def dst(i):
            o, c = sizes[i]
            return out_hbm.at[pl.ds(base + o, c)]

        def stage(i, b):
            return buf.at[sid, b, pl.ds(0, sizes[i][1])]

        g = [None] * nchunk
        s = [None] * nchunk
        for i in range(min(_NBUF, nchunk)):
            g[i] = pltpu.async_copy(src(i), stage(i, i), gsem.at[i])
        for i in range(nchunk):
            b = i % _NBUF
            g[i].wait()
            s[i] = pltpu.async_copy(stage(i, b), dst(i), ssem.at[b])
            nxt = i + 1
            if _NBUF <= nxt < nchunk:
                bn = nxt % _NBUF
                s[nxt - _NBUF].wait()
                g[nxt] = pltpu.async_copy(src(nxt), stage(nxt, bn), gsem.at[bn])
        for i in range(max(0, nchunk - _NBUF), nchunk):
            s[i].wait()

    return copy_k(pe)[None]


# speedup vs baseline: 1.0075x; 1.0075x over previous
import functools

import jax
from jax import lax
from jax.experimental import pallas as pl
from jax.experimental.pallas import tpu as pltpu
from jax.experimental.pallas import tpu_sc as plsc

_BUF_ROWS = 40
_NBUF = 3


def kernel(x, pe):
    seq_len = x.shape[1]
    n_emb = pe.shape[1]
    info = plsc.get_sparse_core_info()
    nc, ns = info.num_cores, info.num_subcores
    nw = nc * ns
    rows_per_w = seq_len // nw

    sizes = []
    off = 0
    while off < rows_per_w:
        c = min(_BUF_ROWS, rows_per_w - off)
        sizes.append((off, c))
        off += c
    nchunk = len(sizes)

    @functools.partial(
        pl.kernel,
        mesh=plsc.VectorSubcoreMesh(core_axis_name="c", subcore_axis_name="s"),
        out_type=jax.ShapeDtypeStruct((seq_len, n_emb), pe.dtype),
        scratch_types=[
            pltpu.VMEM_SHARED((ns, _NBUF, _BUF_ROWS, n_emb), pe.dtype),
            pltpu.SemaphoreType.DMA((_NBUF,)),
            pltpu.SemaphoreType.DMA((_NBUF,)),
        ],
    )
    def copy_k(pe_hbm, out_hbm, buf, gsem, ssem):
        sid = lax.axis_index("s")
        wid = sid * nc + lax.axis_index("c")
        base = wid * rows_per_w

        def src(i):
            o, c = sizes[i]
            return pe_hbm.at[pl.ds(base + o, c)]

        def dst(i):
            o, c = sizes[i]
            return out_hbm.at[pl.ds(base + o, c)]

        def stage(i, b):
            return buf.at[sid, b, pl.ds(0, sizes[i][1])]

        g = [None] * nchunk
        s = [None] * nchunk
        for i in range(min(_NBUF, nchunk)):
            g[i] = pltpu.async_copy(src(i), stage(i, i), gsem.at[i])
        for i in range(nchunk):
            b = i % _NBUF
            g[i].wait()
            s[i] = pltpu.async_copy(stage(i, b), dst(i), ssem.at[b])
            nxt = i + 1
            if _NBUF <= nxt < nchunk:
                bn = nxt % _NBUF
                s[nxt - _NBUF].wait()
                g[nxt] = pltpu.async_copy(src(nxt), stage(nxt, bn), gsem.at[bn])
        for i in range(max(0, nchunk - _NBUF), nchunk):
            s[i].wait()

    return copy_k(pe)[None]
